# baseline (speedup 1.0000x reference)
import jax
import jax.numpy as jnp
from jax import lax
from jax.experimental import pallas as pl
from jax.experimental.pallas import tpu as pltpu
from jax.experimental.pallas import tpu_sc as plsc

_N_FIELDS = 26
_VOCAB = 50
_EMBED_DIM = 32
_PAD = 32


def _body(sample_hbm, table_hbm, out_hbm, idx_v, rows_v, sem):
    c = lax.axis_index("c")
    s = lax.axis_index("s")

    @pl.when(jnp.logical_and(c == 0, s == 0))
    def _():
        pltpu.sync_copy(sample_hbm, idx_v)
        for j in range(2):
            lane = lax.iota(jnp.int32, 16) + (j * 16)
            vals = idx_v[pl.ds(j * 16, 16)]
            idx_v[pl.ds(j * 16, 16)] = jnp.where(
                lane < _N_FIELDS, lane * _VOCAB + vals, 0
            )
        pltpu.async_copy(table_hbm.at[idx_v], rows_v, sem).wait()
        pltpu.sync_copy(rows_v.at[pl.ds(0, _N_FIELDS)], out_hbm)


_gather = pl.kernel(
    _body,
    out_type=jax.ShapeDtypeStruct((_N_FIELDS, _EMBED_DIM), jnp.float32),
    mesh=plsc.VectorSubcoreMesh(core_axis_name="c", subcore_axis_name="s"),
    scratch_types=[
        pltpu.VMEM((_PAD,), jnp.int32),
        pltpu.VMEM((_PAD, _EMBED_DIM), jnp.float32),
        pltpu.SemaphoreType.DMA,
    ],
    compiler_params=pltpu.CompilerParams(use_tc_tiling_on_sc=False),
)


@jax.jit
def kernel(sample, tables):
    sample_p = jnp.concatenate(
        [sample.astype(jnp.int32), jnp.zeros((_PAD - _N_FIELDS,), jnp.int32)]
    )
    table_flat = tables.reshape(_N_FIELDS * _VOCAB, _EMBED_DIM)
    out = _gather(sample_p, table_flat)
    return out.reshape(-1)


# speedup vs baseline: 1.0756x; 1.0756x over previous
import jax
import jax.numpy as jnp
from jax import lax
from jax.experimental import pallas as pl
from jax.experimental.pallas import tpu as pltpu
from jax.experimental.pallas import tpu_sc as plsc

_N_FIELDS = 26
_VOCAB = 50
_EMBED_DIM = 32
_PAD = 32


def _body(sample_hbm, table_hbm, out_hbm, idx_v, rows_v, sem):
    c = lax.axis_index("c")
    s = lax.axis_index("s")

    @pl.when(jnp.logical_and(c == 0, s == 0))
    def _():
        pltpu.sync_copy(sample_hbm, idx_v)
        for j in range(2):
            lane = lax.iota(jnp.int32, 16) + (j * 16)
            vals = idx_v[pl.ds(j * 16, 16)]
            idx_v[pl.ds(j * 16, 16)] = jnp.where(
                lane < _N_FIELDS, lane * _VOCAB + vals, 0
            )
        pltpu.async_copy(table_hbm.at[idx_v], rows_v, sem).wait()
        pltpu.sync_copy(rows_v.at[pl.ds(0, _N_FIELDS)], out_hbm)


_gather = pl.kernel(
    _body,
    out_type=jax.ShapeDtypeStruct((_N_FIELDS, _EMBED_DIM), jnp.float32),
    mesh=plsc.VectorSubcoreMesh(
        core_axis_name="c", subcore_axis_name="s", num_cores=1, num_subcores=1
    ),
    scratch_types=[
        pltpu.VMEM((_PAD,), jnp.int32),
        pltpu.VMEM((_PAD, _EMBED_DIM), jnp.float32),
        pltpu.SemaphoreType.DMA,
    ],
    compiler_params=pltpu.CompilerParams(use_tc_tiling_on_sc=False),
)


@jax.jit
def kernel(sample, tables):
    sample_p = jnp.concatenate(
        [sample.astype(jnp.int32), jnp.zeros((_PAD - _N_FIELDS,), jnp.int32)]
    )
    table_flat = tables.reshape(_N_FIELDS * _VOCAB, _EMBED_DIM)
    out = _gather(sample_p, table_flat)
    return out.reshape(-1)


# speedup vs baseline: 1.1359x; 1.0561x over previous
import jax
import jax.numpy as jnp
from jax import lax
from jax.experimental import pallas as pl
from jax.experimental.pallas import tpu as pltpu
from jax.experimental.pallas import tpu_sc as plsc

_N_FIELDS = 26
_VOCAB = 50
_EMBED_DIM = 32


def _body(sample_hbm, table_hbm, out_hbm, idx_s, sem):
    pltpu.sync_copy(sample_hbm, idx_s)
    copies = []
    for i in range(_N_FIELDS):
        rid = i * _VOCAB + idx_s[i]
        copies.append(
            pltpu.make_async_copy(table_hbm.at[rid], out_hbm.at[i], sem)
        )
        copies[-1].start()
    for c in copies:
        c.wait()


_gather = pl.kernel(
    _body,
    out_type=jax.ShapeDtypeStruct((_N_FIELDS, _EMBED_DIM), jnp.float32),
    mesh=plsc.ScalarSubcoreMesh(axis_name="c", num_cores=1),
    scratch_types=[
        pltpu.SMEM((_N_FIELDS,), jnp.int32),
        pltpu.SemaphoreType.DMA,
    ],
    compiler_params=pltpu.CompilerParams(use_tc_tiling_on_sc=False),
)


@jax.jit
def kernel(sample, tables):
    table_flat = tables.reshape(_N_FIELDS * _VOCAB, _EMBED_DIM)
    out = _gather(sample, table_flat)
    return out.reshape(-1)
